# baseline (speedup 1.0000x reference)
import functools

import jax
import jax.numpy as jnp
from jax import lax
from jax.experimental import pallas as pl
from jax.experimental.pallas import tpu as pltpu
from jax.experimental.pallas import tpu_sc as plsc

NW = 32
CHUNK = 128
LANES = 16
NBR = 5


def _emb_kernel(ids_hbm, tok_hbm, pos_hbm, out_hbm, idx_v, rows_v, pos_v,
                gsem, psem, osem):
    batch, seq = ids_hbm.shape
    pos_per_w = seq // NW
    n_pos_chunks = pos_per_w // CHUNK
    n_units = n_pos_chunks * batch

    sid = lax.axis_index("s")
    wid = sid * 2 + lax.axis_index("c")
    col0 = wid * pos_per_w

    pos_load = pltpu.async_copy(pos_hbm.at[pl.ds(col0, pos_per_w)], pos_v,
                                psem)
    pltpu.sync_copy(ids_hbm.at[:, pl.ds(col0, pos_per_w)], idx_v)
    pos_load.wait()

    units = [(c, b) for c in range(n_pos_chunks) for b in range(batch)]
    g = [None] * n_units
    o = [None] * n_units

    def fill_and_fire(u):
        c, b = units[u]
        rv = rows_v.at[u % NBR]

        @plsc.parallel_loop(0, CHUNK, step=1, unroll=4)
        def fill_body(i):
            for j in range(0, 128, LANES):
                sl = pl.ds(j, LANES)
                rv[i, sl] = pos_v[c * CHUNK + i, sl]

        idx = idx_v.at[b, pl.ds(c * CHUNK, CHUNK)]
        g[u] = pltpu.async_copy(tok_hbm.at[idx], rv, gsem, add=True)

    for u in range(min(NBR - 1, n_units)):
        fill_and_fire(u)

    for u in range(n_units):
        if u + NBR - 1 < n_units:
            if u >= 1:
                o[u - 1].wait()
            fill_and_fire(u + NBR - 1)
        g[u].wait()
        c, b = units[u]
        o[u] = pltpu.async_copy(
            rows_v.at[u % NBR], out_hbm.at[b, pl.ds(col0 + c * CHUNK, CHUNK)],
            osem)

    for u in range(max(0, n_units - NBR), n_units):
        o[u].wait()


@functools.partial(jax.jit, static_argnums=())
def kernel(token_ids, tok_table, pos_table):
    b, s = token_ids.shape
    d = tok_table.shape[1]
    assert s % (NW * CHUNK) == 0 and d == 128

    ids = token_ids.astype(jnp.int32)
    pos_per_w = s // NW

    mesh = plsc.VectorSubcoreMesh(core_axis_name="c", subcore_axis_name="s")
    run = pl.kernel(
        _emb_kernel,
        out_type=jax.ShapeDtypeStruct((b, s, d), jnp.float32),
        mesh=mesh,
        scratch_types=[
            pltpu.VMEM((b, pos_per_w), jnp.int32),
            pltpu.VMEM((NBR, CHUNK, 128), jnp.float32),
            pltpu.VMEM((pos_per_w, 128), jnp.float32),
            pltpu.SemaphoreType.DMA,
            pltpu.SemaphoreType.DMA,
            pltpu.SemaphoreType.DMA,
        ],
    )
    return run(ids, tok_table, pos_table)


# speedup vs baseline: 1.0438x; 1.0438x over previous
import functools

import jax
import jax.numpy as jnp
from jax import lax
from jax.experimental import pallas as pl
from jax.experimental.pallas import tpu as pltpu
from jax.experimental.pallas import tpu_sc as plsc

NW = 32
CHUNK = 128
LANES = 16
NBR = 5


def _emb_kernel(ids_hbm, tok_hbm, pos_hbm, out_hbm, idx_v, rows_v, pos_v,
                gsem, psem, osem):
    batch, seq = ids_hbm.shape
    pos_per_w = seq // NW
    n_pos_chunks = pos_per_w // CHUNK
    n_units = n_pos_chunks * batch

    sid = lax.axis_index("s")
    wid = sid * 2 + lax.axis_index("c")
    col0 = wid * pos_per_w

    pos_load = [
        pltpu.async_copy(pos_hbm.at[pl.ds(col0 + c * CHUNK, CHUNK)],
                         pos_v.at[pl.ds(c * CHUNK, CHUNK)], psem)
        for c in range(n_pos_chunks)
    ]
    pos_ready = [False] * n_pos_chunks
    pltpu.sync_copy(ids_hbm.at[:, pl.ds(col0, pos_per_w)], idx_v)

    units = [(c, b) for c in range(n_pos_chunks) for b in range(batch)]
    g = [None] * n_units
    o = [None] * n_units

    def fill_and_fire(u):
        c, b = units[u]
        if not pos_ready[c]:
            pos_load[c].wait()
            pos_ready[c] = True
        rv = rows_v.at[u % NBR]

        @plsc.parallel_loop(0, CHUNK, step=1, unroll=2)
        def fill_body(i):
            for j in range(0, 128, LANES):
                sl = pl.ds(j, LANES)
                rv[i, sl] = pos_v[c * CHUNK + i, sl]

        idx = idx_v.at[b, pl.ds(c * CHUNK, CHUNK)]
        g[u] = pltpu.async_copy(tok_hbm.at[idx], rv, gsem, add=True)

    for u in range(min(NBR - 1, n_units)):
        fill_and_fire(u)

    for u in range(n_units):
        if u + NBR - 1 < n_units:
            if u >= 1:
                o[u - 1].wait()
            fill_and_fire(u + NBR - 1)
        g[u].wait()
        c, b = units[u]
        o[u] = pltpu.async_copy(
            rows_v.at[u % NBR], out_hbm.at[b, pl.ds(col0 + c * CHUNK, CHUNK)],
            osem)

    for u in range(max(0, n_units - NBR), n_units):
        o[u].wait()


@functools.partial(jax.jit, static_argnums=())
def kernel(token_ids, tok_table, pos_table):
    b, s = token_ids.shape
    d = tok_table.shape[1]
    assert s % (NW * CHUNK) == 0 and d == 128

    ids = token_ids.astype(jnp.int32)
    pos_per_w = s // NW

    mesh = plsc.VectorSubcoreMesh(core_axis_name="c", subcore_axis_name="s")
    run = pl.kernel(
        _emb_kernel,
        out_type=jax.ShapeDtypeStruct((b, s, d), jnp.float32),
        mesh=mesh,
        scratch_types=[
            pltpu.VMEM((b, pos_per_w), jnp.int32),
            pltpu.VMEM((NBR, CHUNK, 128), jnp.float32),
            pltpu.VMEM((pos_per_w, 128), jnp.float32),
            pltpu.SemaphoreType.DMA,
            pltpu.SemaphoreType.DMA,
            pltpu.SemaphoreType.DMA,
        ],
    )
    return run(ids, tok_table, pos_table)
